# baseline (speedup 1.0000x reference)
import functools

import jax
import jax.numpy as jnp
from jax import lax
from jax.experimental import pallas as pl
from jax.experimental.pallas import tpu as pltpu
from jax.experimental.pallas import tpu_sc as plsc

_VOCAB = 1000000
_EMBED = 64
_MAXLEN = 200
_LABELS = 64
_BATCH = 4096

_NC, _NS = 2, 16
_NW = _NC * _NS
_BPW = _BATCH // _NW
_C0, _C1 = 104, 96

_NB = 4000
_HALF = _VOCAB // 2


def _proj_body(tlo_ref, thi_ref, w_ref, o_ref):
    w = w_ref[...]
    dims = (((1,), (1,)), ((), ()))
    o_ref[:, 0:_EMBED] = lax.dot_general(
        tlo_ref[...], w, dims,
        preferred_element_type=jnp.float32,
        precision=lax.Precision.HIGHEST)
    o_ref[:, _EMBED:2 * _EMBED] = lax.dot_general(
        thi_ref[...], w, dims,
        preferred_element_type=jnp.float32,
        precision=lax.Precision.HIGHEST)


def _project(table, fc_w):
    nsteps = _HALF // _NB
    return pl.pallas_call(
        _proj_body,
        grid=(nsteps,),
        in_specs=[
            pl.BlockSpec((_NB, _EMBED), lambda i: (i, 0)),
            pl.BlockSpec((_NB, _EMBED), lambda i, n=nsteps: (i + n, 0)),
            pl.BlockSpec((_LABELS, _EMBED), lambda i: (0, 0)),
        ],
        out_specs=pl.BlockSpec((_NB, 2 * _EMBED), lambda i: (i, 0)),
        out_shape=jax.ShapeDtypeStruct((_HALF, 2 * _EMBED), jnp.float32),
    )(table, table, fc_w)


def _make_mean_kernel():
    mesh = plsc.VectorSubcoreMesh(core_axis_name="c", subcore_axis_name="s")

    @functools.partial(
        pl.kernel,
        out_type=jax.ShapeDtypeStruct((_BATCH, _LABELS), jnp.float32),
        mesh=mesh,
        compiler_params=pltpu.CompilerParams(use_tc_tiling_on_sc=False),
        scratch_types=[
            pltpu.VMEM((_BPW, _MAXLEN), jnp.int32),
            pltpu.VMEM((2, _MAXLEN, _LABELS), jnp.float32),
            pltpu.VMEM((_BPW, _LABELS), jnp.float32),
            pltpu.VMEM((_LABELS,), jnp.float32),
            pltpu.SemaphoreType.DMA,
            pltpu.SemaphoreType.DMA,
        ],
    )
    def mean_kernel(x_hbm, p_hbm, b_hbm, out_hbm,
                    idx_v, rows_v, out_v, b_v, sem0, sem1):
        wid = lax.axis_index("s") * _NC + lax.axis_index("c")
        base = wid * _BPW
        pltpu.sync_copy(b_hbm, b_v)
        pltpu.sync_copy(x_hbm.at[pl.ds(base, _BPW)], idx_v)

        inv_len = jnp.float32(1.0 / _MAXLEN)
        bias = tuple(b_v[pl.ds(c * 16, 16)] for c in range(4))

        def fire(row, buf, sem):
            pltpu.async_copy(
                p_hbm.at[idx_v.at[row, pl.ds(0, _C0)]],
                rows_v.at[buf, pl.ds(0, _C0)], sem)
            pltpu.async_copy(
                p_hbm.at[idx_v.at[row, pl.ds(_C0, _C1)]],
                rows_v.at[buf, pl.ds(_C0, _C1)], sem)

        def wait(buf, sem):
            pltpu.make_async_copy(
                p_hbm.at[idx_v.at[0, pl.ds(0, _C0)]],
                rows_v.at[buf, pl.ds(0, _C0)], sem).wait()
            pltpu.make_async_copy(
                p_hbm.at[idx_v.at[0, pl.ds(_C0, _C1)]],
                rows_v.at[buf, pl.ds(_C0, _C1)], sem).wait()

        def accum_store(i, buf):
            def row_body(rr, accs):
                r = rr * 8
                for u in range(8):
                    accs = tuple(
                        accs[c] + rows_v[buf, r + u, pl.ds(c * 16, 16)]
                        for c in range(4)
                    )
                return accs

            accs = tuple(jnp.zeros((16,), jnp.float32) for _ in range(4))
            accs = lax.fori_loop(0, _MAXLEN // 8, row_body, accs)
            for c in range(4):
                out_v[i, pl.ds(c * 16, 16)] = accs[c] * inv_len + bias[c]

        fire(0, 0, sem0)

        def pair_body(ii, carry):
            i0 = ii * 2
            fire(i0 + 1, 1, sem1)
            wait(0, sem0)
            accum_store(i0, 0)
            nxt = jnp.where(i0 + 2 < _BPW, i0 + 2, 0)
            fire(nxt, 0, sem0)
            wait(1, sem1)
            accum_store(i0 + 1, 1)
            return carry

        lax.fori_loop(0, _BPW // 2, pair_body, 0)
        wait(0, sem0)
        pltpu.sync_copy(out_v, out_hbm.at[pl.ds(base, _BPW)])

    return mean_kernel


_MEAN_KERNEL = _make_mean_kernel()


def kernel(x, table, fc_w, fc_b):
    p_packed = _project(table, fc_w)
    p_rows = p_packed.reshape(_VOCAB, _LABELS)
    x2 = jnp.where(x < _HALF, 2 * x, 2 * x - (_VOCAB - 1))
    return _MEAN_KERNEL(x2, p_rows, fc_b)


# speedup vs baseline: 1.9539x; 1.9539x over previous
import functools

import jax
import jax.numpy as jnp
from jax import lax
from jax.experimental import pallas as pl
from jax.experimental.pallas import tpu as pltpu
from jax.experimental.pallas import tpu_sc as plsc

_VOCAB = 1000000
_EMBED = 64
_MAXLEN = 200
_LABELS = 64
_BATCH = 4096

_NC, _NS = 2, 16
_NW = _NC * _NS
_BPW = _BATCH // _NW
_C0, _C1 = 104, 96

_W = 1920
_NBLK = -(-_VOCAB // _W)
_NPAIR = -(-_NBLK // 2)
_PROWS = _NPAIR * _W


def _proj_body(tlo_ref, thi_ref, w_ref, o_ref):
    w = w_ref[...]
    dims = (((0,), (1,)), ((), ()))
    o_ref[:, 0:_EMBED] = lax.dot_general(
        tlo_ref[...], w, dims, preferred_element_type=jnp.float32)
    o_ref[:, _EMBED:2 * _EMBED] = lax.dot_general(
        thi_ref[...], w, dims, preferred_element_type=jnp.float32)


def _project(table_t, fc_w):
    return pl.pallas_call(
        _proj_body,
        grid=(_NPAIR,),
        in_specs=[
            pl.BlockSpec((_EMBED, _W), lambda i: (0, 2 * i)),
            pl.BlockSpec(
                (_EMBED, _W),
                lambda i: (0, jnp.minimum(2 * i + 1, _NBLK - 1))),
            pl.BlockSpec((_LABELS, _EMBED), lambda i: (0, 0)),
        ],
        out_specs=pl.BlockSpec((_W, 2 * _EMBED), lambda i: (i, 0)),
        out_shape=jax.ShapeDtypeStruct((_PROWS, 2 * _EMBED), jnp.float32),
    )(table_t, table_t, fc_w)


def _make_mean_kernel():
    mesh = plsc.VectorSubcoreMesh(core_axis_name="c", subcore_axis_name="s")

    @functools.partial(
        pl.kernel,
        out_type=jax.ShapeDtypeStruct((_BATCH, _LABELS), jnp.float32),
        mesh=mesh,
        compiler_params=pltpu.CompilerParams(use_tc_tiling_on_sc=False),
        scratch_types=[
            pltpu.VMEM((_BPW, _MAXLEN), jnp.int32),
            pltpu.VMEM((2, _MAXLEN, _LABELS), jnp.float32),
            pltpu.VMEM((_BPW, _LABELS), jnp.float32),
            pltpu.VMEM((_LABELS,), jnp.float32),
            pltpu.SemaphoreType.DMA,
            pltpu.SemaphoreType.DMA,
        ],
    )
    def mean_kernel(x_hbm, p_hbm, b_hbm, out_hbm,
                    idx_v, rows_v, out_v, b_v, sem0, sem1):
        wid = lax.axis_index("s") * _NC + lax.axis_index("c")
        base = wid * _BPW
        pltpu.sync_copy(b_hbm, b_v)
        pltpu.sync_copy(x_hbm.at[pl.ds(base, _BPW)], idx_v)

        inv_len = jnp.float32(1.0 / _MAXLEN)
        bias = tuple(b_v[pl.ds(c * 16, 16)] for c in range(4))

        def fire(row, buf, sem):
            pltpu.async_copy(
                p_hbm.at[idx_v.at[row, pl.ds(0, _C0)]],
                rows_v.at[buf, pl.ds(0, _C0)], sem)
            pltpu.async_copy(
                p_hbm.at[idx_v.at[row, pl.ds(_C0, _C1)]],
                rows_v.at[buf, pl.ds(_C0, _C1)], sem)

        def wait(buf, sem):
            pltpu.make_async_copy(
                p_hbm.at[idx_v.at[0, pl.ds(0, _C0)]],
                rows_v.at[buf, pl.ds(0, _C0)], sem).wait()
            pltpu.make_async_copy(
                p_hbm.at[idx_v.at[0, pl.ds(_C0, _C1)]],
                rows_v.at[buf, pl.ds(_C0, _C1)], sem).wait()

        def accum_store(i, buf):
            def row_body(rr, accs):
                r = rr * 8
                for u in range(8):
                    accs = tuple(
                        accs[c] + rows_v[buf, r + u, pl.ds(c * 16, 16)]
                        for c in range(4)
                    )
                return accs

            accs = tuple(jnp.zeros((16,), jnp.float32) for _ in range(4))
            accs = lax.fori_loop(0, _MAXLEN // 8, row_body, accs)
            for c in range(4):
                out_v[i, pl.ds(c * 16, 16)] = accs[c] * inv_len + bias[c]

        fire(0, 0, sem0)

        def pair_body(ii, carry):
            i0 = ii * 2
            fire(i0 + 1, 1, sem1)
            wait(0, sem0)
            accum_store(i0, 0)
            nxt = jnp.where(i0 + 2 < _BPW, i0 + 2, 0)
            fire(nxt, 0, sem0)
            wait(1, sem1)
            accum_store(i0 + 1, 1)
            return carry

        lax.fori_loop(0, _BPW // 2, pair_body, 0)
        wait(0, sem0)
        pltpu.sync_copy(out_v, out_hbm.at[pl.ds(base, _BPW)])

    return mean_kernel


_MEAN_KERNEL = _make_mean_kernel()


def kernel(x, table, fc_w, fc_b):
    p_packed = _project(table.T, fc_w)
    p_rows = p_packed.reshape(2 * _PROWS, _LABELS)
    b = x // _W
    s = x - b * _W
    x2 = (((b >> 1) * _W + s) << 1) + (b & 1)
    return _MEAN_KERNEL(x2, p_rows, fc_b)


# speedup vs baseline: 2.9857x; 1.5281x over previous
import functools

import jax
import jax.numpy as jnp
from jax import lax
from jax.experimental import pallas as pl
from jax.experimental.pallas import tpu as pltpu
from jax.experimental.pallas import tpu_sc as plsc

_VOCAB = 1000000
_EMBED = 64
_MAXLEN = 200
_LABELS = 64
_BATCH = 4096

_NC, _NS = 2, 16
_NW = _NC * _NS
_BPW = _BATCH // _NW
_C0, _C1 = 104, 96

_W = 7680
_NBLK = -(-_VOCAB // _W)
_NPAIR = -(-_NBLK // 2)
_PROWS = _NPAIR * _W


def _proj_body(tlo_ref, thi_ref, w_ref, o_ref):
    t2 = jnp.concatenate([tlo_ref[...], thi_ref[...]], axis=0)
    o_ref[...] = lax.dot_general(
        t2, w_ref[...], (((0,), (0,)), ((), ())),
        preferred_element_type=jnp.float32)


def _project(table_t, w_blk):
    return pl.pallas_call(
        _proj_body,
        grid=(_NPAIR,),
        in_specs=[
            pl.BlockSpec((_EMBED, _W), lambda i: (0, 2 * i)),
            pl.BlockSpec(
                (_EMBED, _W),
                lambda i: (0, jnp.minimum(2 * i + 1, _NBLK - 1))),
            pl.BlockSpec((2 * _EMBED, 2 * _EMBED), lambda i: (0, 0)),
        ],
        out_specs=pl.BlockSpec((_W, 2 * _EMBED), lambda i: (i, 0)),
        out_shape=jax.ShapeDtypeStruct((_PROWS, 2 * _EMBED), jnp.float32),
    )(table_t, table_t, w_blk)


def _make_mean_kernel():
    mesh = plsc.VectorSubcoreMesh(core_axis_name="c", subcore_axis_name="s")

    @functools.partial(
        pl.kernel,
        out_type=jax.ShapeDtypeStruct((_BATCH, _LABELS), jnp.float32),
        mesh=mesh,
        compiler_params=pltpu.CompilerParams(use_tc_tiling_on_sc=False),
        scratch_types=[
            pltpu.VMEM((_BPW, _MAXLEN), jnp.int32),
            pltpu.VMEM((2, _MAXLEN, _LABELS), jnp.float32),
            pltpu.VMEM((_BPW, _LABELS), jnp.float32),
            pltpu.VMEM((_LABELS,), jnp.float32),
            pltpu.SemaphoreType.DMA,
            pltpu.SemaphoreType.DMA,
        ],
    )
    def mean_kernel(x_hbm, p_hbm, b_hbm, out_hbm,
                    idx_v, rows_v, out_v, b_v, sem0, sem1):
        wid = lax.axis_index("s") * _NC + lax.axis_index("c")
        base = wid * _BPW
        pltpu.sync_copy(b_hbm, b_v)
        pltpu.sync_copy(x_hbm.at[pl.ds(base, _BPW)], idx_v)

        inv_len = jnp.float32(1.0 / _MAXLEN)
        bias = tuple(b_v[pl.ds(c * 16, 16)] for c in range(4))

        def fire(row, buf, sem):
            pltpu.async_copy(
                p_hbm.at[idx_v.at[row, pl.ds(0, _C0)]],
                rows_v.at[buf, pl.ds(0, _C0)], sem)
            pltpu.async_copy(
                p_hbm.at[idx_v.at[row, pl.ds(_C0, _C1)]],
                rows_v.at[buf, pl.ds(_C0, _C1)], sem)

        def wait(buf, sem):
            pltpu.make_async_copy(
                p_hbm.at[idx_v.at[0, pl.ds(0, _C0)]],
                rows_v.at[buf, pl.ds(0, _C0)], sem).wait()
            pltpu.make_async_copy(
                p_hbm.at[idx_v.at[0, pl.ds(_C0, _C1)]],
                rows_v.at[buf, pl.ds(_C0, _C1)], sem).wait()

        def accum_store(i, buf):
            def row_body(rr, accs):
                r = rr * 8
                for u in range(8):
                    accs = tuple(
                        accs[c] + rows_v[buf, r + u, pl.ds(c * 16, 16)]
                        for c in range(4)
                    )
                return accs

            accs = tuple(jnp.zeros((16,), jnp.float32) for _ in range(4))
            accs = lax.fori_loop(0, _MAXLEN // 8, row_body, accs)
            for c in range(4):
                out_v[i, pl.ds(c * 16, 16)] = accs[c] * inv_len + bias[c]

        fire(0, 0, sem0)

        def pair_body(ii, carry):
            i0 = ii * 2
            fire(i0 + 1, 1, sem1)
            wait(0, sem0)
            accum_store(i0, 0)
            nxt = jnp.where(i0 + 2 < _BPW, i0 + 2, 0)
            fire(nxt, 0, sem0)
            wait(1, sem1)
            accum_store(i0 + 1, 1)
            return carry

        lax.fori_loop(0, _BPW // 2, pair_body, 0)
        wait(0, sem0)
        pltpu.sync_copy(out_v, out_hbm.at[pl.ds(base, _BPW)])

    return mean_kernel


_MEAN_KERNEL = _make_mean_kernel()


def kernel(x, table, fc_w, fc_b):
    wt = fc_w.T
    z = jnp.zeros((_EMBED, _EMBED), jnp.float32)
    w_blk = jnp.block([[wt, z], [z, wt]])
    p_packed = _project(table.T, w_blk)
    p_rows = p_packed.reshape(2 * _PROWS, _LABELS)
    b = x // _W
    s = x - b * _W
    x2 = (((b >> 1) * _W + s) << 1) + (b & 1)
    return _MEAN_KERNEL(x2, p_rows, fc_b)


# speedup vs baseline: 3.4091x; 1.1418x over previous
import functools

import jax
import jax.numpy as jnp
from jax import lax
from jax.experimental import pallas as pl
from jax.experimental.pallas import tpu as pltpu
from jax.experimental.pallas import tpu_sc as plsc

_VOCAB = 1000000
_EMBED = 64
_MAXLEN = 200
_LABELS = 64
_BATCH = 4096

_NC, _NS = 2, 16
_NW = _NC * _NS
_BPW = _BATCH // _NW
_C0, _C1 = 104, 96

_W = 7680
_NBLK = -(-_VOCAB // _W)
_NPAIR = -(-_NBLK // 2)
_PROWS = _NPAIR * _W


def _proj_body(tlo_ref, thi_ref, w_ref, o_ref):
    t2 = jnp.concatenate([tlo_ref[...], thi_ref[...]], axis=0)
    o_ref[...] = lax.dot_general(
        t2, w_ref[...], (((0,), (0,)), ((), ())),
        preferred_element_type=jnp.float32)


def _project(table_t, w_blk):
    return pl.pallas_call(
        _proj_body,
        grid=(_NPAIR,),
        in_specs=[
            pl.BlockSpec((_EMBED, _W), lambda i: (0, 2 * i)),
            pl.BlockSpec(
                (_EMBED, _W),
                lambda i: (0, jnp.minimum(2 * i + 1, _NBLK - 1))),
            pl.BlockSpec((2 * _EMBED, 2 * _EMBED), lambda i: (0, 0)),
        ],
        out_specs=pl.BlockSpec((_W, 2 * _EMBED), lambda i: (i, 0)),
        out_shape=jax.ShapeDtypeStruct((_PROWS, 2 * _EMBED), jnp.float32),
    )(table_t, table_t, w_blk)


def _make_mean_kernel():
    mesh = plsc.VectorSubcoreMesh(core_axis_name="c", subcore_axis_name="s")

    @functools.partial(
        pl.kernel,
        out_type=jax.ShapeDtypeStruct((_BATCH, _LABELS), jnp.float32),
        mesh=mesh,
        compiler_params=pltpu.CompilerParams(use_tc_tiling_on_sc=False),
        scratch_types=[
            pltpu.VMEM((_BPW, _MAXLEN), jnp.int32),
            pltpu.VMEM((4, _MAXLEN, _LABELS), jnp.float32),
            pltpu.VMEM((_BPW, _LABELS), jnp.float32),
            pltpu.VMEM((_LABELS,), jnp.float32),
            pltpu.SemaphoreType.DMA,
            pltpu.SemaphoreType.DMA,
            pltpu.SemaphoreType.DMA,
            pltpu.SemaphoreType.DMA,
        ],
    )
    def mean_kernel(x_hbm, p_hbm, b_hbm, out_hbm,
                    idx_v, rows_v, out_v, b_v, sem0, sem1, sem2, sem3):
        wid = lax.axis_index("s") * _NC + lax.axis_index("c")
        base = wid * _BPW
        pltpu.sync_copy(b_hbm, b_v)
        pltpu.sync_copy(x_hbm.at[pl.ds(base, _BPW)], idx_v)

        inv_len = jnp.float32(1.0 / _MAXLEN)
        bias = tuple(b_v[pl.ds(c * 16, 16)] for c in range(4))

        def fire(row, buf, sem):
            pltpu.async_copy(
                p_hbm.at[idx_v.at[row, pl.ds(0, _C0)]],
                rows_v.at[buf, pl.ds(0, _C0)], sem)
            pltpu.async_copy(
                p_hbm.at[idx_v.at[row, pl.ds(_C0, _C1)]],
                rows_v.at[buf, pl.ds(_C0, _C1)], sem)

        def wait(buf, sem):
            pltpu.make_async_copy(
                p_hbm.at[idx_v.at[0, pl.ds(0, _C0)]],
                rows_v.at[buf, pl.ds(0, _C0)], sem).wait()
            pltpu.make_async_copy(
                p_hbm.at[idx_v.at[0, pl.ds(_C0, _C1)]],
                rows_v.at[buf, pl.ds(_C0, _C1)], sem).wait()

        def accum_store(i, buf):
            def row_body(rr, accs):
                r = rr * 8
                for u in range(8):
                    accs = tuple(
                        accs[c] + rows_v[buf, r + u, pl.ds(c * 16, 16)]
                        for c in range(4)
                    )
                return accs

            accs = tuple(jnp.zeros((16,), jnp.float32) for _ in range(4))
            accs = lax.fori_loop(0, _MAXLEN // 8, row_body, accs)
            for c in range(4):
                out_v[i, pl.ds(c * 16, 16)] = accs[c] * inv_len + bias[c]

        sems = (sem0, sem1, sem2, sem3)
        for u in range(4):
            fire(u, u, sems[u])

        def quad_body(ii, carry):
            i0 = ii * 4
            for u in range(4):
                wait(u, sems[u])
                accum_store(i0 + u, u)
                nxt = jnp.where(i0 + u + 4 < _BPW, i0 + u + 4, 0)
                fire(nxt, u, sems[u])
            return carry

        lax.fori_loop(0, _BPW // 4, quad_body, 0)
        for u in range(4):
            wait(u, sems[u])
        pltpu.sync_copy(out_v, out_hbm.at[pl.ds(base, _BPW)])

    return mean_kernel


_MEAN_KERNEL = _make_mean_kernel()


def kernel(x, table, fc_w, fc_b):
    wt = fc_w.T
    z = jnp.zeros((_EMBED, _EMBED), jnp.float32)
    w_blk = jnp.block([[wt, z], [z, wt]])
    p_packed = _project(table.T, w_blk)
    p_rows = p_packed.reshape(2 * _PROWS, _LABELS)
    b = x // _W
    s = x - b * _W
    x2 = (((b >> 1) * _W + s) << 1) + (b & 1)
    return _MEAN_KERNEL(x2, p_rows, fc_b)
